# baseline (speedup 1.0000x reference)
import functools

import jax
import jax.numpy as jnp
from jax import lax
from jax.experimental import pallas as pl
from jax.experimental.pallas import tpu as pltpu
from jax.experimental.pallas import tpu_sc as plsc

_NN_SCALE = 1999853.335557038
_H = 4
_DH = 32
_D = 16
_G = 17
_NB = _G * _D
_ROWS = 304


def _elu(x):
    return jnp.where(x > 0, x, jnp.exp(x) - 1.0)


def _leaky(x):
    return jnp.where(x > 0, x, 0.2 * x)


def _ln(x, g, b):
    mu = jnp.mean(x, -1, keepdims=True)
    var = jnp.mean((x - mu) ** 2, -1, keepdims=True)
    return (x - mu) / jnp.sqrt(var + 1e-5) * g + b


def _sc_gather(q, adj2d, nf2d):
    n_nodes, din = nf2d.shape
    mesh = plsc.VectorSubcoreMesh(core_axis_name="c", subcore_axis_name="s")

    @functools.partial(
        pl.kernel,
        out_type=jax.ShapeDtypeStruct((_ROWS, din), jnp.float32),
        mesh=mesh,
        scratch_types=[
            pltpu.VMEM((16,), jnp.int32),
            pltpu.VMEM((1, 128), jnp.int32),
            pltpu.VMEM((_D,), jnp.int32),
            pltpu.VMEM((_D, 128), jnp.int32),
            pltpu.VMEM((_ROWS,), jnp.int32),
            pltpu.VMEM((_ROWS, din), jnp.float32),
            pltpu.SemaphoreType.DMA,
        ],
    )
    def k(q_hbm, adj_hbm, nf_hbm, out_hbm, qpad_v, adjq_v, nbr_v, adj1_v,
          idx2_v, x_v, sem):
        cid = lax.axis_index("c")
        sid = lax.axis_index("s")

        @pl.when(jnp.logical_and(cid == 0, sid == 0))
        def _():
            qpad_v[...] = jnp.zeros((16,), jnp.int32)
            pltpu.sync_copy(q_hbm, qpad_v.at[pl.ds(0, 1)])
            pltpu.async_copy(adj_hbm.at[qpad_v.at[pl.ds(0, 1)]], adjq_v,
                             sem).wait()
            nbr = adjq_v[0, pl.ds(0, _D)]
            nbr_v[...] = nbr
            pltpu.async_copy(adj_hbm.at[nbr_v], adj1_v, sem).wait()
            for j in range(_D):
                idx2_v[pl.ds(j * _D, _D)] = adj1_v[j, pl.ds(0, _D)]
            idx2_v[pl.ds(_D * _D, _D)] = nbr
            idx2_v[pl.ds(_NB, _D)] = nbr
            idx2_v[pl.ds(_NB + _D, 16)] = qpad_v[...]
            pltpu.async_copy(nf_hbm.at[idx2_v], x_v, sem).wait()
            pltpu.sync_copy(x_v, out_hbm)

    return k(q, adj2d, nf2d)


def _dot(a, b):
    return jnp.dot(a, b, preferred_element_type=jnp.float32)


def _dot_hi(a, b):
    return jnp.dot(a, b, precision=lax.Precision.HIGHEST,
                   preferred_element_type=jnp.float32)


def _tc_body(x_ref, wi_ref, bi_ref,
             wq1_ref, wv1_ref, ws1_ref, wd1_ref, lw1_ref, lb1_ref, g1_ref, b1_ref,
             wq2_ref, wv2_ref, ws2_ref, wd2_ref, lw2_ref, lb2_ref, g2_ref, b2_ref,
             f0w_ref, f0b_ref, f1w_ref, f1b_ref, f2w_ref, f2b_ref,
             o_ref):
    x0 = _elu(_dot(x_ref[...], wi_ref[...]) + bi_ref[...])

    hq = _dot(x0, wq1_ref[...])
    hv = _dot(x0, wv1_ref[...])
    ssb = _dot_hi(hq, ws1_ref[...])
    sdb = _dot_hi(hq, wd1_ref[...])
    ss_self = ssb[_NB:_NB + _G, :].reshape(_G, 1, 128)
    sd_nb = sdb[:_NB, :].reshape(_G, _D, 128)
    e = _leaky(ss_self + sd_nb)
    e = e - jnp.max(e, axis=1, keepdims=True)
    ex = jnp.exp(e)
    alpha = ex / (jnp.sum(ex, axis=1, keepdims=True) + 1e-9)
    vnb = hv[:_NB, :].reshape(_G, _D, 128)
    att = jnp.sum(alpha * vnb, axis=1)
    a1 = _elu(_dot(att, lw1_ref[...]) + lb1_ref[...])
    x_self = x0[_NB:_NB + _G, :]
    x1 = _ln(a1 + x_self, g1_ref[...], b1_ref[...])

    hq2 = _dot(x1, wq2_ref[...])
    hv2 = _dot(x1, wv2_ref[...])
    ssb2 = _dot_hi(hq2, ws2_ref[...])
    sdb2 = _dot_hi(hq2, wd2_ref[...])
    e2 = _leaky(ssb2[_D:_D + 1, :] + sdb2[:_D, :])
    e2 = e2 - jnp.max(e2, axis=0, keepdims=True)
    ex2 = jnp.exp(e2)
    alpha2 = ex2 / (jnp.sum(ex2, axis=0, keepdims=True) + 1e-9)
    att2 = jnp.sum(alpha2 * hv2[:_D, :], axis=0, keepdims=True)
    a2 = _elu(_dot(att2, lw2_ref[...]) + lb2_ref[...])
    x2 = _ln(a2 + x1[_D:_D + 1, :], g2_ref[...], b2_ref[...])

    v = _elu(_dot(x2, f0w_ref[...]) + f0b_ref[...])
    v = _elu(_dot(v, f1w_ref[...]) + f1b_ref[...])
    v = _elu(_dot(v, f2w_ref[...]) + f2b_ref[...])
    o_ref[...] = v * _NN_SCALE


def _prep_layer(lp):
    wq = jnp.transpose(lp['Wq'], (1, 0, 2)).reshape(128, _H * _DH)
    wv = jnp.transpose(lp['Wv'], (1, 0, 2)).reshape(128, _H * _DH)
    lane = jnp.arange(_H * _DH)
    blk = (lane[:, None] // _DH == lane[None, :] // _DH).astype(jnp.float32)
    ws = lp['a_src'].reshape(-1)[:, None] * blk
    wd = lp['a_dst'].reshape(-1)[:, None] * blk
    return (wq, wv, ws, wd, lp['lin_W'], lp['lin_b'].reshape(1, -1),
            lp['ln_g'].reshape(1, -1), lp['ln_b'].reshape(1, -1))


def kernel(node_features, query_idxs, masks, adj, sim_results, params):
    del masks, sim_results
    nf2d = node_features[0]
    adj2d = jnp.zeros((10000, 128), jnp.int32)
    q = query_idxs.astype(jnp.int32)

    x = _sc_gather(q, adj2d, nf2d)

    l1 = _prep_layer(params['layers'][0])
    l2 = _prep_layer(params['layers'][1])
    (f0w, f0b), (f1w, f1b), (f2w, f2b) = params['final']
    args = (x, params['init_W'], params['init_b'].reshape(1, -1),
            *l1, *l2,
            f0w, f0b.reshape(1, -1), f1w, f1b.reshape(1, -1),
            f2w, f2b.reshape(1, -1))

    out = pl.pallas_call(
        _tc_body,
        out_shape=jax.ShapeDtypeStruct((1, 32), jnp.float32),
    )(*args)
    return out


# speedup vs baseline: 1.6825x; 1.6825x over previous
import functools

import jax
import jax.numpy as jnp
from jax import lax
from jax.experimental import pallas as pl
from jax.experimental.pallas import tpu as pltpu
from jax.experimental.pallas import tpu_sc as plsc

_NN_SCALE = 1999853.335557038
_H = 4
_DH = 32
_D = 16
_G = 17
_NB = _G * _D
_ROWS = 304


def _elu(x):
    return jnp.where(x > 0, x, jnp.exp(x) - 1.0)


def _leaky(x):
    return jnp.where(x > 0, x, 0.2 * x)


def _ln(x, g, b):
    mu = jnp.mean(x, -1, keepdims=True)
    var = jnp.mean((x - mu) ** 2, -1, keepdims=True)
    return (x - mu) / jnp.sqrt(var + 1e-5) * g + b


def _sc_gather(q, adj2d, nf2d):
    n_nodes, din = nf2d.shape
    mesh = plsc.VectorSubcoreMesh(core_axis_name="c", subcore_axis_name="s")

    @functools.partial(
        pl.kernel,
        out_type=jax.ShapeDtypeStruct((_ROWS, din), jnp.float32),
        mesh=mesh,
        scratch_types=[
            pltpu.VMEM((16,), jnp.int32),
            pltpu.VMEM((1, _D), jnp.int32),
            pltpu.VMEM((_D, _D), jnp.int32),
            pltpu.VMEM((_ROWS,), jnp.int32),
            pltpu.VMEM((_ROWS, din), jnp.float32),
            pltpu.SemaphoreType.DMA,
        ],
    )
    def k(q_hbm, adj_hbm, nf_hbm, out_hbm, qpad_v, adjq_v, adj1_v,
          idx2_v, x_v, sem):
        cid = lax.axis_index("c")
        sid = lax.axis_index("s")

        @pl.when(jnp.logical_and(cid == 0, sid == 0))
        def _():
            qpad_v[...] = jnp.zeros((16,), jnp.int32)
            pltpu.sync_copy(q_hbm, qpad_v.at[pl.ds(0, 1)])
            qvec = qpad_v[...]
            qs = qvec[0]
            pltpu.sync_copy(adj_hbm.at[pl.ds(qs, 1)], adjq_v)
            nbr = adjq_v[0]
            copies = [
                pltpu.async_copy(adj_hbm.at[pl.ds(nbr[j], 1)],
                                 adj1_v.at[pl.ds(j, 1)], sem)
                for j in range(_D)
            ]
            for c in copies:
                c.wait()
            for j in range(_D):
                idx2_v[pl.ds(j * _D, _D)] = adj1_v[j]
            idx2_v[pl.ds(_D * _D, _D)] = nbr
            idx2_v[pl.ds(_NB, _D)] = nbr
            idx2_v[pl.ds(_NB + _D, 16)] = qvec
            pltpu.async_copy(nf_hbm.at[idx2_v], x_v, sem).wait()
            pltpu.sync_copy(x_v, out_hbm)

    return k(q, adj2d, nf2d)


def _dot(a, b):
    return jnp.dot(a, b, preferred_element_type=jnp.float32)


def _dot_hi(a, b):
    return jnp.dot(a, b, precision=lax.Precision.HIGHEST,
                   preferred_element_type=jnp.float32)


def _tc_body(x_ref, wi_ref, bi_ref,
             wq1_ref, wv1_ref, ws1_ref, wd1_ref, lw1_ref, lb1_ref, g1_ref, b1_ref,
             wq2_ref, wv2_ref, ws2_ref, wd2_ref, lw2_ref, lb2_ref, g2_ref, b2_ref,
             f0w_ref, f0b_ref, f1w_ref, f1b_ref, f2w_ref, f2b_ref,
             o_ref):
    x0 = _elu(_dot(x_ref[...], wi_ref[...]) + bi_ref[...])

    hq = _dot(x0, wq1_ref[...])
    hv = _dot(x0, wv1_ref[...])
    ssb = _dot_hi(hq, ws1_ref[...])
    sdb = _dot_hi(hq, wd1_ref[...])
    ss_self = ssb[_NB:_NB + _G, :].reshape(_G, 1, 128)
    sd_nb = sdb[:_NB, :].reshape(_G, _D, 128)
    e = _leaky(ss_self + sd_nb)
    e = e - jnp.max(e, axis=1, keepdims=True)
    ex = jnp.exp(e)
    alpha = ex / (jnp.sum(ex, axis=1, keepdims=True) + 1e-9)
    vnb = hv[:_NB, :].reshape(_G, _D, 128)
    att = jnp.sum(alpha * vnb, axis=1)
    a1 = _elu(_dot(att, lw1_ref[...]) + lb1_ref[...])
    x_self = x0[_NB:_NB + _G, :]
    x1 = _ln(a1 + x_self, g1_ref[...], b1_ref[...])

    hq2 = _dot(x1, wq2_ref[...])
    hv2 = _dot(x1, wv2_ref[...])
    ssb2 = _dot_hi(hq2, ws2_ref[...])
    sdb2 = _dot_hi(hq2, wd2_ref[...])
    e2 = _leaky(ssb2[_D:_D + 1, :] + sdb2[:_D, :])
    e2 = e2 - jnp.max(e2, axis=0, keepdims=True)
    ex2 = jnp.exp(e2)
    alpha2 = ex2 / (jnp.sum(ex2, axis=0, keepdims=True) + 1e-9)
    att2 = jnp.sum(alpha2 * hv2[:_D, :], axis=0, keepdims=True)
    a2 = _elu(_dot(att2, lw2_ref[...]) + lb2_ref[...])
    x2 = _ln(a2 + x1[_D:_D + 1, :], g2_ref[...], b2_ref[...])

    v = _elu(_dot(x2, f0w_ref[...]) + f0b_ref[...])
    v = _elu(_dot(v, f1w_ref[...]) + f1b_ref[...])
    v = _elu(_dot(v, f2w_ref[...]) + f2b_ref[...])
    o_ref[...] = v * _NN_SCALE


def _prep_layer(lp):
    wq = jnp.transpose(lp['Wq'], (1, 0, 2)).reshape(128, _H * _DH)
    wv = jnp.transpose(lp['Wv'], (1, 0, 2)).reshape(128, _H * _DH)
    lane = jnp.arange(_H * _DH)
    blk = (lane[:, None] // _DH == lane[None, :] // _DH).astype(jnp.float32)
    ws = lp['a_src'].reshape(-1)[:, None] * blk
    wd = lp['a_dst'].reshape(-1)[:, None] * blk
    return (wq, wv, ws, wd, lp['lin_W'], lp['lin_b'].reshape(1, -1),
            lp['ln_g'].reshape(1, -1), lp['ln_b'].reshape(1, -1))


def kernel(node_features, query_idxs, masks, adj, sim_results, params):
    del masks, sim_results
    nf2d = node_features[0]
    adj2d = adj[0].astype(jnp.int32)
    q = query_idxs.astype(jnp.int32)

    x = _sc_gather(q, adj2d, nf2d)

    l1 = _prep_layer(params['layers'][0])
    l2 = _prep_layer(params['layers'][1])
    (f0w, f0b), (f1w, f1b), (f2w, f2b) = params['final']
    args = (x, params['init_W'], params['init_b'].reshape(1, -1),
            *l1, *l2,
            f0w, f0b.reshape(1, -1), f1w, f1b.reshape(1, -1),
            f2w, f2b.reshape(1, -1))

    out = pl.pallas_call(
        _tc_body,
        out_shape=jax.ShapeDtypeStruct((1, 32), jnp.float32),
    )(*args)
    return out


# speedup vs baseline: 1.7668x; 1.0501x over previous
import functools

import jax
import jax.numpy as jnp
from jax import lax
from jax.experimental import pallas as pl
from jax.experimental.pallas import tpu as pltpu
from jax.experimental.pallas import tpu_sc as plsc

_NN_SCALE = 1999853.335557038
_H = 4
_DH = 32
_D = 16
_G = 17
_NB = _G * _D
_ROWS = 304


def _elu(x):
    return jnp.where(x > 0, x, jnp.exp(x) - 1.0)


def _leaky(x):
    return jnp.where(x > 0, x, 0.2 * x)


def _ln(x, g, b):
    mu = jnp.mean(x, -1, keepdims=True)
    var = jnp.mean((x - mu) ** 2, -1, keepdims=True)
    return (x - mu) / jnp.sqrt(var + 1e-5) * g + b


def _sc_gather(q, adj2d, nf2d):
    n_nodes, din = nf2d.shape
    mesh = plsc.VectorSubcoreMesh(core_axis_name="c", subcore_axis_name="s",
                                  num_cores=1)

    @functools.partial(
        pl.kernel,
        out_type=jax.ShapeDtypeStruct((_ROWS, din), jnp.float32),
        mesh=mesh,
        scratch_types=[
            pltpu.VMEM((16,), jnp.int32),
            pltpu.VMEM((1, _D), jnp.int32),
            pltpu.VMEM((_D, _D), jnp.int32),
            pltpu.VMEM((_ROWS,), jnp.int32),
            pltpu.VMEM((_ROWS, din), jnp.float32),
            pltpu.SemaphoreType.DMA,
        ],
    )
    def k(q_hbm, adj_hbm, nf_hbm, out_hbm, qpad_v, adjq_v, adj1_v,
          idx2_v, x_v, sem):
        cid = lax.axis_index("c")
        sid = lax.axis_index("s")

        @pl.when(jnp.logical_and(cid == 0, sid == 0))
        def _():
            qpad_v[...] = jnp.zeros((16,), jnp.int32)
            pltpu.sync_copy(q_hbm, qpad_v.at[pl.ds(0, 1)])
            qvec = qpad_v[...]
            qs = qvec[0]
            pltpu.sync_copy(adj_hbm.at[pl.ds(qs, 1)], adjq_v)
            nbr = adjq_v[0]
            copies = [
                pltpu.async_copy(adj_hbm.at[pl.ds(nbr[j], 1)],
                                 adj1_v.at[pl.ds(j, 1)], sem)
                for j in range(_D)
            ]
            for c in copies:
                c.wait()
            for j in range(_D):
                idx2_v[pl.ds(j * _D, _D)] = adj1_v[j]
            idx2_v[pl.ds(_D * _D, _D)] = nbr
            idx2_v[pl.ds(_NB, _D)] = nbr
            idx2_v[pl.ds(_NB + _D, 16)] = qvec
            pltpu.async_copy(nf_hbm.at[idx2_v], x_v, sem).wait()
            pltpu.sync_copy(x_v, out_hbm)

    return k(q, adj2d, nf2d)


def _dot(a, b):
    return jnp.dot(a, b, preferred_element_type=jnp.float32)


def _dot_hi(a, b):
    return jnp.dot(a, b, precision=lax.Precision.HIGHEST,
                   preferred_element_type=jnp.float32)


def _tc_body(x_ref, wi_ref, bi_ref,
             wq1_ref, wv1_ref, ws1_ref, wd1_ref, lw1_ref, lb1_ref, g1_ref, b1_ref,
             wq2_ref, wv2_ref, ws2_ref, wd2_ref, lw2_ref, lb2_ref, g2_ref, b2_ref,
             f0w_ref, f0b_ref, f1w_ref, f1b_ref, f2w_ref, f2b_ref,
             o_ref):
    x0 = _elu(_dot(x_ref[...], wi_ref[...]) + bi_ref[...])

    hq = _dot(x0, wq1_ref[...])
    hv = _dot(x0, wv1_ref[...])
    ssb = _dot_hi(hq, ws1_ref[...])
    sdb = _dot_hi(hq, wd1_ref[...])
    ss_self = ssb[_NB:_NB + _G, :].reshape(_G, 1, 128)
    sd_nb = sdb[:_NB, :].reshape(_G, _D, 128)
    e = _leaky(ss_self + sd_nb)
    e = e - jnp.max(e, axis=1, keepdims=True)
    ex = jnp.exp(e)
    alpha = ex / (jnp.sum(ex, axis=1, keepdims=True) + 1e-9)
    vnb = hv[:_NB, :].reshape(_G, _D, 128)
    att = jnp.sum(alpha * vnb, axis=1)
    a1 = _elu(_dot(att, lw1_ref[...]) + lb1_ref[...])
    x_self = x0[_NB:_NB + _G, :]
    x1 = _ln(a1 + x_self, g1_ref[...], b1_ref[...])

    hq2 = _dot(x1, wq2_ref[...])
    hv2 = _dot(x1, wv2_ref[...])
    ssb2 = _dot_hi(hq2, ws2_ref[...])
    sdb2 = _dot_hi(hq2, wd2_ref[...])
    e2 = _leaky(ssb2[_D:_D + 1, :] + sdb2[:_D, :])
    e2 = e2 - jnp.max(e2, axis=0, keepdims=True)
    ex2 = jnp.exp(e2)
    alpha2 = ex2 / (jnp.sum(ex2, axis=0, keepdims=True) + 1e-9)
    att2 = jnp.sum(alpha2 * hv2[:_D, :], axis=0, keepdims=True)
    a2 = _elu(_dot(att2, lw2_ref[...]) + lb2_ref[...])
    x2 = _ln(a2 + x1[_D:_D + 1, :], g2_ref[...], b2_ref[...])

    v = _elu(_dot(x2, f0w_ref[...]) + f0b_ref[...])
    v = _elu(_dot(v, f1w_ref[...]) + f1b_ref[...])
    v = _elu(_dot(v, f2w_ref[...]) + f2b_ref[...])
    o_ref[...] = v * _NN_SCALE


def _prep_layer(lp):
    wq = jnp.transpose(lp['Wq'], (1, 0, 2)).reshape(128, _H * _DH)
    wv = jnp.transpose(lp['Wv'], (1, 0, 2)).reshape(128, _H * _DH)
    lane = jnp.arange(_H * _DH)
    blk = (lane[:, None] // _DH == lane[None, :] // _DH).astype(jnp.float32)
    ws = lp['a_src'].reshape(-1)[:, None] * blk
    wd = lp['a_dst'].reshape(-1)[:, None] * blk
    return (wq, wv, ws, wd, lp['lin_W'], lp['lin_b'].reshape(1, -1),
            lp['ln_g'].reshape(1, -1), lp['ln_b'].reshape(1, -1))


def kernel(node_features, query_idxs, masks, adj, sim_results, params):
    del masks, sim_results
    nf2d = node_features[0]
    adj2d = adj[0].astype(jnp.int32)
    q = query_idxs.astype(jnp.int32)

    x = _sc_gather(q, adj2d, nf2d)

    l1 = _prep_layer(params['layers'][0])
    l2 = _prep_layer(params['layers'][1])
    (f0w, f0b), (f1w, f1b), (f2w, f2b) = params['final']
    args = (x, params['init_W'], params['init_b'].reshape(1, -1),
            *l1, *l2,
            f0w, f0b.reshape(1, -1), f1w, f1b.reshape(1, -1),
            f2w, f2b.reshape(1, -1))

    out = pl.pallas_call(
        _tc_body,
        out_shape=jax.ShapeDtypeStruct((1, 32), jnp.float32),
    )(*args)
    return out


# speedup vs baseline: 1.7672x; 1.0002x over previous
import functools

import jax
import jax.numpy as jnp
from jax import lax
from jax.experimental import pallas as pl
from jax.experimental.pallas import tpu as pltpu
from jax.experimental.pallas import tpu_sc as plsc

_NN_SCALE = 1999853.335557038
_H = 4
_DH = 32
_D = 16
_G = 17
_NB = _G * _D
_ROWS = 304


def _elu(x):
    return jnp.where(x > 0, x, jnp.exp(x) - 1.0)


def _leaky(x):
    return jnp.where(x > 0, x, 0.2 * x)


def _ln(x, g, b):
    mu = jnp.mean(x, -1, keepdims=True)
    var = jnp.mean((x - mu) ** 2, -1, keepdims=True)
    return (x - mu) / jnp.sqrt(var + 1e-5) * g + b


def _sc_gather(q, adj2d, nf2d):
    n_nodes, din = nf2d.shape
    mesh = plsc.VectorSubcoreMesh(core_axis_name="c", subcore_axis_name="s",
                                  num_cores=1, num_subcores=1)

    @functools.partial(
        pl.kernel,
        out_type=jax.ShapeDtypeStruct((_ROWS, din), jnp.float32),
        mesh=mesh,
        scratch_types=[
            pltpu.VMEM((16,), jnp.int32),
            pltpu.VMEM((1, _D), jnp.int32),
            pltpu.VMEM((_D, _D), jnp.int32),
            pltpu.VMEM((_ROWS,), jnp.int32),
            pltpu.VMEM((_ROWS, din), jnp.float32),
            pltpu.SemaphoreType.DMA,
        ],
    )
    def k(q_hbm, adj_hbm, nf_hbm, out_hbm, qpad_v, adjq_v, adj1_v,
          idx2_v, x_v, sem):
        cid = lax.axis_index("c")
        sid = lax.axis_index("s")

        @pl.when(jnp.logical_and(cid == 0, sid == 0))
        def _():
            qpad_v[...] = jnp.zeros((16,), jnp.int32)
            pltpu.sync_copy(q_hbm, qpad_v.at[pl.ds(0, 1)])
            qvec = qpad_v[...]
            qs = qvec[0]
            pltpu.sync_copy(adj_hbm.at[pl.ds(qs, 1)], adjq_v)
            nbr = adjq_v[0]
            copies = [
                pltpu.async_copy(adj_hbm.at[pl.ds(nbr[j], 1)],
                                 adj1_v.at[pl.ds(j, 1)], sem)
                for j in range(_D)
            ]
            for c in copies:
                c.wait()
            for j in range(_D):
                idx2_v[pl.ds(j * _D, _D)] = adj1_v[j]
            idx2_v[pl.ds(_D * _D, _D)] = nbr
            idx2_v[pl.ds(_NB, _D)] = nbr
            idx2_v[pl.ds(_NB + _D, 16)] = qvec
            pltpu.async_copy(nf_hbm.at[idx2_v], x_v, sem).wait()
            pltpu.sync_copy(x_v, out_hbm)

    return k(q, adj2d, nf2d)


def _dot(a, b):
    return jnp.dot(a, b, preferred_element_type=jnp.float32)


def _dot_hi(a, b):
    return jnp.dot(a, b, precision=lax.Precision.HIGHEST,
                   preferred_element_type=jnp.float32)


def _tc_body(x_ref, wi_ref, bi_ref,
             wq1_ref, wv1_ref, ws1_ref, wd1_ref, lw1_ref, lb1_ref, g1_ref, b1_ref,
             wq2_ref, wv2_ref, ws2_ref, wd2_ref, lw2_ref, lb2_ref, g2_ref, b2_ref,
             f0w_ref, f0b_ref, f1w_ref, f1b_ref, f2w_ref, f2b_ref,
             o_ref):
    x0 = _elu(_dot(x_ref[...], wi_ref[...]) + bi_ref[...])

    hq = _dot(x0, wq1_ref[...])
    hv = _dot(x0, wv1_ref[...])
    ssb = _dot_hi(hq, ws1_ref[...])
    sdb = _dot_hi(hq, wd1_ref[...])
    ss_self = ssb[_NB:_NB + _G, :].reshape(_G, 1, 128)
    sd_nb = sdb[:_NB, :].reshape(_G, _D, 128)
    e = _leaky(ss_self + sd_nb)
    e = e - jnp.max(e, axis=1, keepdims=True)
    ex = jnp.exp(e)
    alpha = ex / (jnp.sum(ex, axis=1, keepdims=True) + 1e-9)
    vnb = hv[:_NB, :].reshape(_G, _D, 128)
    att = jnp.sum(alpha * vnb, axis=1)
    a1 = _elu(_dot(att, lw1_ref[...]) + lb1_ref[...])
    x_self = x0[_NB:_NB + _G, :]
    x1 = _ln(a1 + x_self, g1_ref[...], b1_ref[...])

    hq2 = _dot(x1, wq2_ref[...])
    hv2 = _dot(x1, wv2_ref[...])
    ssb2 = _dot_hi(hq2, ws2_ref[...])
    sdb2 = _dot_hi(hq2, wd2_ref[...])
    e2 = _leaky(ssb2[_D:_D + 1, :] + sdb2[:_D, :])
    e2 = e2 - jnp.max(e2, axis=0, keepdims=True)
    ex2 = jnp.exp(e2)
    alpha2 = ex2 / (jnp.sum(ex2, axis=0, keepdims=True) + 1e-9)
    att2 = jnp.sum(alpha2 * hv2[:_D, :], axis=0, keepdims=True)
    a2 = _elu(_dot(att2, lw2_ref[...]) + lb2_ref[...])
    x2 = _ln(a2 + x1[_D:_D + 1, :], g2_ref[...], b2_ref[...])

    v = _elu(_dot(x2, f0w_ref[...]) + f0b_ref[...])
    v = _elu(_dot(v, f1w_ref[...]) + f1b_ref[...])
    v = _elu(_dot(v, f2w_ref[...]) + f2b_ref[...])
    o_ref[...] = v * _NN_SCALE


def _prep_layer(lp):
    wq = jnp.transpose(lp['Wq'], (1, 0, 2)).reshape(128, _H * _DH)
    wv = jnp.transpose(lp['Wv'], (1, 0, 2)).reshape(128, _H * _DH)
    lane = jnp.arange(_H * _DH)
    blk = (lane[:, None] // _DH == lane[None, :] // _DH).astype(jnp.float32)
    ws = lp['a_src'].reshape(-1)[:, None] * blk
    wd = lp['a_dst'].reshape(-1)[:, None] * blk
    return (wq, wv, ws, wd, lp['lin_W'], lp['lin_b'].reshape(1, -1),
            lp['ln_g'].reshape(1, -1), lp['ln_b'].reshape(1, -1))


def kernel(node_features, query_idxs, masks, adj, sim_results, params):
    del masks, sim_results
    nf2d = node_features[0]
    adj2d = adj[0].astype(jnp.int32)
    q = query_idxs.astype(jnp.int32)

    x = _sc_gather(q, adj2d, nf2d)

    l1 = _prep_layer(params['layers'][0])
    l2 = _prep_layer(params['layers'][1])
    (f0w, f0b), (f1w, f1b), (f2w, f2b) = params['final']
    args = (x, params['init_W'], params['init_b'].reshape(1, -1),
            *l1, *l2,
            f0w, f0b.reshape(1, -1), f1w, f1b.reshape(1, -1),
            f2w, f2b.reshape(1, -1))

    out = pl.pallas_call(
        _tc_body,
        out_shape=jax.ShapeDtypeStruct((1, 32), jnp.float32),
    )(*args)
    return out
